# Initial kernel scaffold; baseline (speedup 1.0000x reference)
import jax
import jax.numpy as jnp
from jax.experimental import pallas as pl


def kernel(industry_ids, emb_table, W, b):
    raise NotImplementedError("write your pallas kernel here")


# speedup vs baseline: 2.8052x; 2.8052x over previous
import functools

import jax
import jax.numpy as jnp
from jax import lax
from jax.experimental import pallas as pl
from jax.experimental.pallas import tpu as pltpu
from jax.experimental.pallas import tpu_sc as plsc

_B = 16384
_H = 50
_V = 1000
_D = 128
_NB = _B * _H

_NC = 2
_NS = 16
_NW = _NC * _NS
_BPW = _NB // _NW
_CHUNK = 256
_NCHUNK = _BPW // _CHUNK


def _transform_body(e_ref, w_ref, b_ref, t_ref):
    prod = lax.dot_general(
        e_ref[...], w_ref[...], (((1,), (1,)), ((), ())),
        preferred_element_type=jnp.float32,
        precision=lax.Precision.HIGHEST)
    t_ref[...] = jnp.maximum(prod + b_ref[...], 0.0)


def _transform_table(emb_table, W, b):
    return pl.pallas_call(
        _transform_body,
        out_shape=jax.ShapeDtypeStruct((_V, _D), jnp.float32),
    )(emb_table, W, b.reshape(1, _D))


def _gather_body(table_hbm, idx_hbm, out_hbm, idx_v, rows_v, sem):
    wid = lax.axis_index("s") * _NC + lax.axis_index("c")
    base = wid * _BPW
    pltpu.sync_copy(idx_hbm.at[pl.ds(base, _BPW)], idx_v)

    def body(i, carry):
        off = pl.multiple_of(i * _CHUNK, _CHUNK)
        pltpu.async_copy(
            table_hbm.at[idx_v.at[pl.ds(off, _CHUNK)]], rows_v, sem).wait()
        pltpu.sync_copy(rows_v, out_hbm.at[pl.ds(base + off, _CHUNK)])
        return carry

    lax.fori_loop(0, _NCHUNK, body, 0)


def _gather(table, idx):
    mesh = plsc.VectorSubcoreMesh(core_axis_name="c", subcore_axis_name="s")
    run = functools.partial(
        pl.kernel,
        mesh=mesh,
        out_type=jax.ShapeDtypeStruct((_NB, _D), jnp.float32),
        scratch_types=[
            pltpu.VMEM((_BPW,), jnp.int32),
            pltpu.VMEM((_CHUNK, _D), jnp.float32),
            pltpu.SemaphoreType.DMA,
        ],
    )(_gather_body)
    return run(table, idx)


def kernel(industry_ids, emb_table, W, b):
    table = _transform_table(emb_table, W, b)
    idx = industry_ids.reshape(_NB).astype(jnp.int32)
    out = _gather(table, idx)
    return out.reshape(_B, _H, _D)


# speedup vs baseline: 2.8287x; 1.0084x over previous
import functools

import jax
import jax.numpy as jnp
from jax import lax
from jax.experimental import pallas as pl
from jax.experimental.pallas import tpu as pltpu
from jax.experimental.pallas import tpu_sc as plsc

_B = 16384
_H = 50
_V = 1000
_D = 128
_NB = _B * _H

_NC = 2
_NS = 16
_NW = _NC * _NS
_BPW = _NB // _NW
_CHUNK = 320
_NCHUNK = _BPW // _CHUNK
_NPAIR = _NCHUNK // 2


def _transform_body(e_ref, w_ref, b_ref, t_ref):
    prod = lax.dot_general(
        e_ref[...], w_ref[...], (((1,), (1,)), ((), ())),
        preferred_element_type=jnp.float32,
        precision=lax.Precision.HIGHEST)
    t_ref[...] = jnp.maximum(prod + b_ref[...], 0.0)


def _transform_table(emb_table, W, b):
    return pl.pallas_call(
        _transform_body,
        out_shape=jax.ShapeDtypeStruct((_V, _D), jnp.float32),
    )(emb_table, W, b.reshape(1, _D))


def _gather_body(table_hbm, idx_hbm, out_hbm, idx_v, buf0, buf1, g0, g1, o0, o1):
    wid = lax.axis_index("s") * _NC + lax.axis_index("c")
    base = wid * _BPW
    pltpu.sync_copy(idx_hbm.at[pl.ds(base, _BPW)], idx_v)

    def idx_sl(i):
        return idx_v.at[pl.ds(pl.multiple_of(i * _CHUNK, _CHUNK), _CHUNK)]

    def out_sl(i):
        return out_hbm.at[pl.ds(base + pl.multiple_of(i * _CHUNK, _CHUNK),
                                _CHUNK)]

    pltpu.async_copy(table_hbm.at[idx_sl(0)], buf0, g0)
    pltpu.async_copy(table_hbm.at[idx_sl(1)], buf1, g1)

    def body(j, carry):
        i = j * 2
        pltpu.make_async_copy(table_hbm.at[idx_sl(i)], buf0, g0).wait()
        pltpu.async_copy(buf0, out_sl(i), o0)
        pltpu.make_async_copy(table_hbm.at[idx_sl(i + 1)], buf1, g1).wait()
        pltpu.async_copy(buf1, out_sl(i + 1), o1)
        pltpu.make_async_copy(buf0, out_sl(i), o0).wait()
        pltpu.async_copy(table_hbm.at[idx_sl(i + 2)], buf0, g0)
        pltpu.make_async_copy(buf1, out_sl(i + 1), o1).wait()
        pltpu.async_copy(table_hbm.at[idx_sl(i + 3)], buf1, g1)
        return carry

    lax.fori_loop(0, _NPAIR - 1, body, 0)

    i = (_NPAIR - 1) * 2
    pltpu.make_async_copy(table_hbm.at[idx_sl(i)], buf0, g0).wait()
    pltpu.sync_copy(buf0, out_sl(i))
    pltpu.make_async_copy(table_hbm.at[idx_sl(i + 1)], buf1, g1).wait()
    pltpu.sync_copy(buf1, out_sl(i + 1))


def _gather(table, idx):
    mesh = plsc.VectorSubcoreMesh(core_axis_name="c", subcore_axis_name="s")
    run = functools.partial(
        pl.kernel,
        mesh=mesh,
        out_type=jax.ShapeDtypeStruct((_NB, _D), jnp.float32),
        scratch_types=[
            pltpu.VMEM((_BPW,), jnp.int32),
            pltpu.VMEM((_CHUNK, _D), jnp.float32),
            pltpu.VMEM((_CHUNK, _D), jnp.float32),
            pltpu.SemaphoreType.DMA,
            pltpu.SemaphoreType.DMA,
            pltpu.SemaphoreType.DMA,
            pltpu.SemaphoreType.DMA,
        ],
    )(_gather_body)
    return run(table, idx)


def kernel(industry_ids, emb_table, W, b):
    table = _transform_table(emb_table, W, b)
    idx = industry_ids.reshape(_NB).astype(jnp.int32)
    out = _gather(table, idx)
    return out.reshape(_B, _H, _D)
